# baseline (speedup 1.0000x reference)
import functools

import jax
import jax.numpy as jnp
from jax import lax
from jax.experimental import pallas as pl
from jax.experimental.pallas import tpu as pltpu
from jax.experimental.pallas import tpu_sc as plsc

PRE_SEQ_LEN = 128
HIDDEN_SIZE = 2048
NUM_LAYERS = 24
EMBED_DIM = 2 * NUM_LAYERS * HIDDEN_SIZE
BATCH = 8
PREFIX_LEN = 128

NB = BATCH * PREFIX_LEN
NW = 32
ROWS = NB // NW
NCHUNK = 64
CD = EMBED_DIM // NCHUNK


def _body(gidx_hbm, oidx_hbm, table_hbm, out_hbm,
          gidx_v, oidx_v, b0, sg0, ss0):
    wid = lax.axis_index("s") * 2 + lax.axis_index("c")
    pltpu.sync_copy(gidx_hbm.at[wid], gidx_v)
    pltpu.sync_copy(oidx_hbm.at[wid], oidx_v)

    @pl.loop(0, NCHUNK)
    def _chunk(c):
        pltpu.async_copy(table_hbm.at[gidx_v.at[c]], b0, sg0).wait()
        pltpu.async_copy(b0, out_hbm.at[oidx_v.at[c]], ss0).wait()


@jax.jit
def _run(gidx, oidx, table2):
    mesh = plsc.VectorSubcoreMesh(core_axis_name="c", subcore_axis_name="s")
    f = pl.kernel(
        _body,
        out_type=jax.ShapeDtypeStruct((NB * NCHUNK, CD), jnp.float32),
        mesh=mesh,
        scratch_types=[
            pltpu.VMEM((NCHUNK, ROWS), jnp.int32),
            pltpu.VMEM((NCHUNK, ROWS), jnp.int32),
            pltpu.VMEM((ROWS, CD), jnp.float32),
            pltpu.SemaphoreType.DMA,
            pltpu.SemaphoreType.DMA,
        ],
    )
    return f(gidx, oidx, table2)


def kernel(prefix, table):
    pf = prefix.reshape(NW, ROWS).astype(jnp.int32)
    chunks = jnp.arange(NCHUNK, dtype=jnp.int32)[None, :, None]
    gidx = pf[:, None, :] * NCHUNK + chunks
    rows = jnp.arange(NB, dtype=jnp.int32).reshape(NW, ROWS)
    oidx = rows[:, None, :] * NCHUNK + chunks
    table2 = table.reshape(PRE_SEQ_LEN * NCHUNK, CD)
    out2 = _run(gidx, oidx, table2)
    return out2.reshape(BATCH, PREFIX_LEN, EMBED_DIM)


# speedup vs baseline: 1.0559x; 1.0559x over previous
import functools

import jax
import jax.numpy as jnp
from jax import lax
from jax.experimental import pallas as pl
from jax.experimental.pallas import tpu as pltpu
from jax.experimental.pallas import tpu_sc as plsc

PRE_SEQ_LEN = 128
HIDDEN_SIZE = 2048
NUM_LAYERS = 24
EMBED_DIM = 2 * NUM_LAYERS * HIDDEN_SIZE
BATCH = 8
PREFIX_LEN = 128

NB = BATCH * PREFIX_LEN
NW = 32
ROWS = NB // NW
NCHUNK = 64
CD = EMBED_DIM // NCHUNK


def _body(gidx_hbm, oidx_hbm, table_hbm, out_hbm,
          gidx_v, oidx_v, b0, b1, sg0, sg1, ss0, ss1):
    wid = lax.axis_index("s") * 2 + lax.axis_index("c")
    pltpu.sync_copy(gidx_hbm.at[wid], gidx_v)
    pltpu.sync_copy(oidx_hbm.at[wid], oidx_v)

    def g_start(c, buf, sem):
        pltpu.async_copy(table_hbm.at[gidx_v.at[c]], buf, sem)

    def g_wait(c, buf, sem):
        pltpu.make_async_copy(table_hbm.at[gidx_v.at[c]], buf, sem).wait()

    def s_start(c, buf, sem):
        pltpu.async_copy(buf, out_hbm.at[oidx_v.at[c]], sem)

    def s_wait(c, buf, sem):
        pltpu.make_async_copy(buf, out_hbm.at[oidx_v.at[c]], sem).wait()

    NG = NCHUNK // 2
    g_start(0, b0, sg0)
    g_wait(0, b0, sg0)
    g_start(1, b1, sg1)
    s_start(0, b0, ss0)
    g_wait(1, b1, sg1)
    s_wait(0, b0, ss0)
    g_start(2, b0, sg0)
    s_start(1, b1, ss1)

    @pl.loop(1, NG)
    def _iter(g):
        c0 = 2 * g
        g_wait(c0, b0, sg0)
        s_wait(c0 - 1, b1, ss1)
        g_start(c0 + 1, b1, sg1)
        s_start(c0, b0, ss0)
        g_wait(c0 + 1, b1, sg1)
        s_wait(c0, b0, ss0)

        @pl.when(g < NG - 1)
        def _():
            g_start(c0 + 2, b0, sg0)

        s_start(c0 + 1, b1, ss1)

    s_wait(NCHUNK - 1, b1, ss1)


@jax.jit
def _run(gidx, oidx, table2):
    mesh = plsc.VectorSubcoreMesh(core_axis_name="c", subcore_axis_name="s")
    f = pl.kernel(
        _body,
        out_type=jax.ShapeDtypeStruct((NB * NCHUNK, CD), jnp.float32),
        mesh=mesh,
        scratch_types=[
            pltpu.VMEM((NCHUNK, ROWS), jnp.int32),
            pltpu.VMEM((NCHUNK, ROWS), jnp.int32),
            pltpu.VMEM((ROWS, CD), jnp.float32),
            pltpu.VMEM((ROWS, CD), jnp.float32),
            pltpu.SemaphoreType.DMA,
            pltpu.SemaphoreType.DMA,
            pltpu.SemaphoreType.DMA,
            pltpu.SemaphoreType.DMA,
        ],
    )
    return f(gidx, oidx, table2)


def kernel(prefix, table):
    pf = prefix.reshape(NW, ROWS).astype(jnp.int32)
    chunks = jnp.arange(NCHUNK, dtype=jnp.int32)[None, :, None]
    gidx = pf[:, None, :] * NCHUNK + chunks
    rows = jnp.arange(NB, dtype=jnp.int32).reshape(NW, ROWS)
    oidx = rows[:, None, :] * NCHUNK + chunks
    table2 = table.reshape(PRE_SEQ_LEN * NCHUNK, CD)
    out2 = _run(gidx, oidx, table2)
    return out2.reshape(BATCH, PREFIX_LEN, EMBED_DIM)


# speedup vs baseline: 1.0993x; 1.0411x over previous
import jax
import jax.numpy as jnp
from jax import lax
from jax.experimental import pallas as pl
from jax.experimental.pallas import tpu as pltpu
from jax.experimental.pallas import tpu_sc as plsc

PRE_SEQ_LEN = 128
HIDDEN_SIZE = 2048
NUM_LAYERS = 24
EMBED_DIM = 2 * NUM_LAYERS * HIDDEN_SIZE
BATCH = 8
PREFIX_LEN = 128

NB = BATCH * PREFIX_LEN
NW = 32
ROWS = NB // NW

RPS = 4
BUF_WORDS = 49152
CD = BUF_WORDS // RPS
NCH = EMBED_DIM // CD
NSTREAM = (ROWS // RPS) * NCH


def _body(gidx_hbm, oidx_hbm, table_hbm, out_hbm,
          gidx_v, oidx_v, b0, b1, sg0, sg1, ss0, ss1):
    wid = lax.axis_index("s") * 2 + lax.axis_index("c")
    pltpu.sync_copy(gidx_hbm.at[wid], gidx_v)
    pltpu.sync_copy(oidx_hbm.at[wid], oidx_v)

    def g_start(c, buf, sem):
        pltpu.async_copy(table_hbm.at[gidx_v.at[c]], buf, sem)

    def g_wait(c, buf, sem):
        pltpu.make_async_copy(table_hbm.at[gidx_v.at[c]], buf, sem).wait()

    def s_start(c, buf, sem):
        pltpu.async_copy(buf, out_hbm.at[oidx_v.at[c]], sem)

    def s_wait(c, buf, sem):
        pltpu.make_async_copy(buf, out_hbm.at[oidx_v.at[c]], sem).wait()

    NG = NSTREAM // 2
    g_start(0, b0, sg0)
    g_wait(0, b0, sg0)
    g_start(1, b1, sg1)
    s_start(0, b0, ss0)
    g_wait(1, b1, sg1)
    s_wait(0, b0, ss0)
    g_start(2, b0, sg0)
    s_start(1, b1, ss1)

    @pl.loop(1, NG)
    def _iter(g):
        c0 = 2 * g
        g_wait(c0, b0, sg0)
        s_wait(c0 - 1, b1, ss1)
        g_start(c0 + 1, b1, sg1)
        s_start(c0, b0, ss0)
        g_wait(c0 + 1, b1, sg1)
        s_wait(c0, b0, ss0)

        @pl.when(g < NG - 1)
        def _():
            g_start(c0 + 2, b0, sg0)

        s_start(c0 + 1, b1, ss1)

    s_wait(NSTREAM - 1, b1, ss1)


@jax.jit
def _run(gidx, oidx, table2):
    mesh = plsc.VectorSubcoreMesh(core_axis_name="c", subcore_axis_name="s")
    f = pl.kernel(
        _body,
        out_type=jax.ShapeDtypeStruct((NB * NCH, CD), jnp.float32),
        mesh=mesh,
        scratch_types=[
            pltpu.VMEM((NSTREAM, RPS), jnp.int32),
            pltpu.VMEM((NSTREAM, RPS), jnp.int32),
            pltpu.VMEM((RPS, CD), jnp.float32),
            pltpu.VMEM((RPS, CD), jnp.float32),
            pltpu.SemaphoreType.DMA,
            pltpu.SemaphoreType.DMA,
            pltpu.SemaphoreType.DMA,
            pltpu.SemaphoreType.DMA,
        ],
    )
    return f(gidx, oidx, table2)


def kernel(prefix, table):
    pf = prefix.reshape(NW, ROWS // RPS, 1, RPS).astype(jnp.int32)
    chunks = jnp.arange(NCH, dtype=jnp.int32)[None, None, :, None]
    gidx = (pf * NCH + chunks).reshape(NW, NSTREAM, RPS)
    rows = jnp.arange(NB, dtype=jnp.int32).reshape(NW, ROWS // RPS, 1, RPS)
    oidx = (rows * NCH + chunks).reshape(NW, NSTREAM, RPS)
    table2 = table.reshape(PRE_SEQ_LEN * NCH, CD)
    out2 = _run(gidx, oidx, table2)
    return out2.reshape(BATCH, PREFIX_LEN, EMBED_DIM)


# speedup vs baseline: 1.1709x; 1.0652x over previous
import jax
import jax.numpy as jnp
from jax import lax
from jax.experimental import pallas as pl
from jax.experimental.pallas import tpu as pltpu
from jax.experimental.pallas import tpu_sc as plsc

PRE_SEQ_LEN = 128
HIDDEN_SIZE = 2048
NUM_LAYERS = 24
EMBED_DIM = 2 * NUM_LAYERS * HIDDEN_SIZE
BATCH = 8
PREFIX_LEN = 128

NB = BATCH * PREFIX_LEN
NW = 32
ROWS = NB // NW

RPS = 1
BUF_WORDS = 49152
CD = BUF_WORDS // RPS
NCH = EMBED_DIM // CD
NSTREAM = (ROWS // RPS) * NCH


def _body(gidx_hbm, oidx_hbm, table_hbm, out_hbm,
          gidx_v, oidx_v, b0, b1, sg0, sg1, ss0, ss1):
    wid = lax.axis_index("s") * 2 + lax.axis_index("c")
    pltpu.sync_copy(gidx_hbm.at[wid], gidx_v)
    pltpu.sync_copy(oidx_hbm.at[wid], oidx_v)

    def g_start(c, buf, sem):
        pltpu.async_copy(table_hbm.at[gidx_v.at[c]], buf, sem)

    def g_wait(c, buf, sem):
        pltpu.make_async_copy(table_hbm.at[gidx_v.at[c]], buf, sem).wait()

    def s_start(c, buf, sem):
        pltpu.async_copy(buf, out_hbm.at[oidx_v.at[c]], sem)

    def s_wait(c, buf, sem):
        pltpu.make_async_copy(buf, out_hbm.at[oidx_v.at[c]], sem).wait()

    NG = NSTREAM // 2
    g_start(0, b0, sg0)
    g_wait(0, b0, sg0)
    g_start(1, b1, sg1)
    s_start(0, b0, ss0)
    g_wait(1, b1, sg1)
    s_wait(0, b0, ss0)
    g_start(2, b0, sg0)
    s_start(1, b1, ss1)

    @pl.loop(1, NG)
    def _iter(g):
        c0 = 2 * g
        g_wait(c0, b0, sg0)
        s_wait(c0 - 1, b1, ss1)
        g_start(c0 + 1, b1, sg1)
        s_start(c0, b0, ss0)
        g_wait(c0 + 1, b1, sg1)
        s_wait(c0, b0, ss0)

        @pl.when(g < NG - 1)
        def _():
            g_start(c0 + 2, b0, sg0)

        s_start(c0 + 1, b1, ss1)

    s_wait(NSTREAM - 1, b1, ss1)


@jax.jit
def _run(gidx, oidx, table2):
    mesh = plsc.VectorSubcoreMesh(core_axis_name="c", subcore_axis_name="s")
    f = pl.kernel(
        _body,
        out_type=jax.ShapeDtypeStruct((NB * NCH, CD), jnp.float32),
        mesh=mesh,
        scratch_types=[
            pltpu.VMEM((NSTREAM, RPS), jnp.int32),
            pltpu.VMEM((NSTREAM, RPS), jnp.int32),
            pltpu.VMEM((RPS, CD), jnp.float32),
            pltpu.VMEM((RPS, CD), jnp.float32),
            pltpu.SemaphoreType.DMA,
            pltpu.SemaphoreType.DMA,
            pltpu.SemaphoreType.DMA,
            pltpu.SemaphoreType.DMA,
        ],
    )
    return f(gidx, oidx, table2)


def kernel(prefix, table):
    pf = prefix.reshape(NW, ROWS // RPS, 1, RPS).astype(jnp.int32)
    chunks = jnp.arange(NCH, dtype=jnp.int32)[None, None, :, None]
    gidx = (pf * NCH + chunks).reshape(NW, NSTREAM, RPS)
    rows = jnp.arange(NB, dtype=jnp.int32).reshape(NW, ROWS // RPS, 1, RPS)
    oidx = (rows * NCH + chunks).reshape(NW, NSTREAM, RPS)
    table2 = table.reshape(PRE_SEQ_LEN * NCH, CD)
    out2 = _run(gidx, oidx, table2)
    return out2.reshape(BATCH, PREFIX_LEN, EMBED_DIM)


# speedup vs baseline: 2.4148x; 2.0623x over previous
import jax
import jax.numpy as jnp
from jax import lax
from jax.experimental import pallas as pl
from jax.experimental.pallas import tpu as pltpu
from jax.experimental.pallas import tpu_sc as plsc

PRE_SEQ_LEN = 128
HIDDEN_SIZE = 2048
NUM_LAYERS = 24
EMBED_DIM = 2 * NUM_LAYERS * HIDDEN_SIZE
BATCH = 8
PREFIX_LEN = 128

NB = BATCH * PREFIX_LEN
NW = 32
ROWS = NB // NW

RPS = 1
BUF_WORDS = 49152
CD = BUF_WORDS // RPS
NCH = EMBED_DIM // CD
NSTREAM = (ROWS // RPS) * NCH


def _body(gidx_hbm, table_hbm, out_hbm, gidx_v, b0, b1, sg0, sg1, ss0, ss1):
    wid = lax.axis_index("s") * 2 + lax.axis_index("c")
    pltpu.sync_copy(gidx_hbm.at[wid], gidx_v)

    def g_start(s, buf, sem):
        col = (s % NCH) * CD
        pltpu.async_copy(table_hbm.at[gidx_v.at[s], pl.ds(col, CD)], buf, sem)

    def g_wait(s, buf, sem):
        col = (s % NCH) * CD
        pltpu.make_async_copy(table_hbm.at[gidx_v.at[s], pl.ds(col, CD)], buf,
                              sem).wait()

    def _dst(s):
        row = wid * ROWS + (s // NCH) * RPS
        col = (s % NCH) * CD
        return out_hbm.at[pl.ds(row, RPS), pl.ds(col, CD)]

    def s_start(s, buf, sem):
        pltpu.async_copy(buf, _dst(s), sem)

    def s_wait(s, buf, sem):
        pltpu.make_async_copy(buf, _dst(s), sem).wait()

    NG = NSTREAM // 2
    g_start(0, b0, sg0)
    g_wait(0, b0, sg0)
    g_start(1, b1, sg1)
    s_start(0, b0, ss0)
    g_wait(1, b1, sg1)
    s_wait(0, b0, ss0)
    g_start(2, b0, sg0)
    s_start(1, b1, ss1)

    @pl.loop(1, NG)
    def _iter(g):
        c0 = 2 * g
        g_wait(c0, b0, sg0)
        s_wait(c0 - 1, b1, ss1)
        g_start(c0 + 1, b1, sg1)
        s_start(c0, b0, ss0)
        g_wait(c0 + 1, b1, sg1)
        s_wait(c0, b0, ss0)

        @pl.when(g < NG - 1)
        def _():
            g_start(c0 + 2, b0, sg0)

        s_start(c0 + 1, b1, ss1)

    s_wait(NSTREAM - 1, b1, ss1)


@jax.jit
def _run(gidx, table):
    mesh = plsc.VectorSubcoreMesh(core_axis_name="c", subcore_axis_name="s")
    f = pl.kernel(
        _body,
        out_type=jax.ShapeDtypeStruct((NB, EMBED_DIM), jnp.float32),
        mesh=mesh,
        scratch_types=[
            pltpu.VMEM((NSTREAM, RPS), jnp.int32),
            pltpu.VMEM((RPS, CD), jnp.float32),
            pltpu.VMEM((RPS, CD), jnp.float32),
            pltpu.SemaphoreType.DMA,
            pltpu.SemaphoreType.DMA,
            pltpu.SemaphoreType.DMA,
            pltpu.SemaphoreType.DMA,
        ],
    )
    return f(gidx, table)


def kernel(prefix, table):
    pf = prefix.reshape(NW, ROWS // RPS, 1, RPS).astype(jnp.int32)
    gidx = jnp.broadcast_to(pf, (NW, ROWS // RPS, NCH, RPS)).reshape(
        NW, NSTREAM, RPS)
    out2 = _run(gidx, table)
    return out2.reshape(BATCH, PREFIX_LEN, EMBED_DIM)
